# baseline (speedup 1.0000x reference)
import functools

import jax
import jax.numpy as jnp
from jax import lax
from jax.experimental import pallas as pl
from jax.experimental.pallas import tpu as pltpu
from jax.experimental.pallas import tpu_sc as plsc

B = 4096
L = 50
D = 64
NC = 2
NS = 16
NW = NC * NS
BPW = B // NW
CH = 8
NCHUNK = BPW // CH
VL = 16
DK = D // VL


def _sc_gather_sum(x0, x1, x2, w0, w1, w2):
    mesh = plsc.VectorSubcoreMesh(core_axis_name="c", subcore_axis_name="s")

    @functools.partial(
        pl.kernel,
        out_type=jax.ShapeDtypeStruct((B, D), jnp.float32),
        mesh=mesh,
        scratch_types=[
            pltpu.VMEM((BPW * L,), jnp.int32),
            pltpu.VMEM((BPW * L,), jnp.int32),
            pltpu.VMEM((BPW * L,), jnp.int32),
            pltpu.VMEM((CH * L, D), jnp.float32),
            pltpu.VMEM((CH * L, D), jnp.float32),
            pltpu.VMEM((BPW, D), jnp.float32),
            pltpu.SemaphoreType.DMA,
            pltpu.SemaphoreType.DMA,
        ],
        compiler_params=pltpu.CompilerParams(use_tc_tiling_on_sc=False),
    )
    def k(x0h, x1h, x2h, w0h, w1h, w2h, out_h,
          idx0, idx1, idx2, rowsA, rowsB, acc, semA, semB):
        wid = lax.axis_index("s") * NC + lax.axis_index("c")
        base = wid * BPW

        pltpu.sync_copy(x0h.at[pl.ds(base * L, BPW * L)], idx0)
        pltpu.sync_copy(x1h.at[pl.ds(base * L, BPW * L)], idx1)
        pltpu.sync_copy(x2h.at[pl.ds(base * L, BPW * L)], idx2)

        for t, (tbl, idxv) in enumerate(((w0h, idx0), (w1h, idx1), (w2h, idx2))):

            def issue(j, buf, sem):
                pltpu.async_copy(
                    tbl.at[idxv.at[pl.ds(j * CH * L, CH * L)]], buf, sem)

            def wait(buf, sem):
                pltpu.make_async_copy(
                    tbl.at[idxv.at[pl.ds(0, CH * L)]], buf, sem).wait()

            def accum(j, buf):
                for c in range(CH):
                    row = j * CH + c
                    init = tuple(buf[c * L, pl.ds(VL * kk, VL)]
                                 for kk in range(DK))

                    def body(l, carry):
                        return tuple(
                            carry[kk] + buf[c * L + l, pl.ds(VL * kk, VL)]
                            for kk in range(DK))

                    sums = lax.fori_loop(1, L, body, init)
                    for kk in range(DK):
                        sl = (row, pl.ds(VL * kk, VL))
                        if t == 0:
                            acc[sl] = sums[kk]
                        else:
                            acc[sl] = acc[sl] + sums[kk]

            issue(0, rowsA, semA)

            @pl.loop(0, NCHUNK - 2, step=2)
            def _(jj):
                issue(jj + 1, rowsB, semB)
                wait(rowsA, semA)
                accum(jj, rowsA)
                issue(jj + 2, rowsA, semA)
                wait(rowsB, semB)
                accum(jj + 1, rowsB)

            issue(NCHUNK - 1, rowsB, semB)
            wait(rowsA, semA)
            accum(NCHUNK - 2, rowsA)
            wait(rowsB, semB)
            accum(NCHUNK - 1, rowsB)

        pltpu.sync_copy(acc, out_h.at[pl.ds(base, BPW), :])

    return k(x0, x1, x2, w0, w1, w2)


RELAYOUT_BLK = 2048


def _tc_relayout(w):
    V = w.shape[0]
    wt = w.T

    def body(wt_ref, o_ref):
        ident = (jax.lax.broadcasted_iota(jnp.int32, (D, D), 0) ==
                 jax.lax.broadcasted_iota(jnp.int32, (D, D), 1)
                 ).astype(jnp.float32)
        tt = jax.lax.dot_general(
            wt_ref[...], ident, (((0,), (0,)), ((), ())),
            preferred_element_type=jnp.float32,
            precision=jax.lax.Precision.HIGHEST)
        t3 = tt.reshape(RELAYOUT_BLK // 2, 2, D)
        a = t3[:, 0, :]
        b = t3[:, 1, :]
        o_ref[...] = jnp.concatenate([a, b], axis=1)

    out = pl.pallas_call(
        body,
        grid=(pl.cdiv(V, RELAYOUT_BLK),),
        in_specs=[pl.BlockSpec((D, RELAYOUT_BLK), lambda j: (0, j))],
        out_specs=pl.BlockSpec((RELAYOUT_BLK // 2, 2 * D), lambda j: (j, 0)),
        out_shape=jax.ShapeDtypeStruct((V // 2, 2 * D), jnp.float32),
    )(wt)
    return out.reshape(V, D)


def _tc_mlp_softmax(e_sum, w_h, b_h, w_o, b_o):
    def body(e_ref, wh_ref, bh_ref, wo_ref, bo_ref, o_ref):
        wo = wo_ref[...]
        wc = jnp.dot(wh_ref[...], wo, preferred_element_type=jnp.float32)
        bc = jnp.dot(bh_ref[...], wo, preferred_element_type=jnp.float32) \
            + bo_ref[...]
        e_avg = e_ref[...] * (1.0 / (3 * L))
        logits = jnp.dot(e_avg, wc, preferred_element_type=jnp.float32) + bc
        m = jnp.max(logits, axis=1, keepdims=True)
        ex = jnp.exp(logits - m)
        o_ref[...] = ex / jnp.sum(ex, axis=1, keepdims=True)

    return pl.pallas_call(
        body,
        out_shape=jax.ShapeDtypeStruct((B, b_o.shape[-1]), jnp.float32),
    )(e_sum, w_h, b_h, w_o, b_o)


def kernel(x_0, x_1, x_2, W_word, W_2gram, W_3gram, W_h, b_h, W_o, b_o):
    x_0 = x_0.astype(jnp.int32).reshape(-1)
    x_1 = x_1.astype(jnp.int32).reshape(-1)
    x_2 = x_2.astype(jnp.int32).reshape(-1)
    W_word, W_2gram, W_3gram = [
        _tc_relayout(w) for w in (W_word, W_2gram, W_3gram)]
    e_sum = _sc_gather_sum(x_0, x_1, x_2, W_word, W_2gram, W_3gram)
    return _tc_mlp_softmax(e_sum, W_h.astype(jnp.float32),
                           b_h.reshape(1, -1).astype(jnp.float32),
                           W_o.astype(jnp.float32),
                           b_o.reshape(1, -1).astype(jnp.float32))


# speedup vs baseline: 1.1352x; 1.1352x over previous
import functools

import jax
import jax.numpy as jnp
from jax import lax
from jax.experimental import pallas as pl
from jax.experimental.pallas import tpu as pltpu
from jax.experimental.pallas import tpu_sc as plsc

B = 4096
L = 50
D = 64
NC = 2
NS = 16
NW = NC * NS
BPW = B // NW
CH = 8
NCHUNK = BPW // CH
VL = 16
DK = D // VL


def _sc_gather_sum(x0, x1, x2, w0, w1, w2):
    mesh = plsc.VectorSubcoreMesh(core_axis_name="c", subcore_axis_name="s")

    @functools.partial(
        pl.kernel,
        out_type=jax.ShapeDtypeStruct((B, D), jnp.float32),
        mesh=mesh,
        scratch_types=[
            pltpu.VMEM((BPW * L,), jnp.int32),
            pltpu.VMEM((BPW * L,), jnp.int32),
            pltpu.VMEM((BPW * L,), jnp.int32),
            pltpu.VMEM((CH * L, D), jnp.float32),
            pltpu.VMEM((CH * L, D), jnp.float32),
            pltpu.VMEM((BPW, D), jnp.float32),
            pltpu.SemaphoreType.DMA,
            pltpu.SemaphoreType.DMA,
        ],
        compiler_params=pltpu.CompilerParams(use_tc_tiling_on_sc=False),
    )
    def k(x0h, x1h, x2h, w0h, w1h, w2h, out_h,
          idx0, idx1, idx2, rowsA, rowsB, acc, semA, semB):
        wid = lax.axis_index("s") * NC + lax.axis_index("c")
        base = wid * BPW

        pltpu.sync_copy(x0h.at[pl.ds(base * L, BPW * L)], idx0)
        pltpu.sync_copy(x1h.at[pl.ds(base * L, BPW * L)], idx1)
        pltpu.sync_copy(x2h.at[pl.ds(base * L, BPW * L)], idx2)

        for t, (tbl, idxv) in enumerate(((w0h, idx0), (w1h, idx1), (w2h, idx2))):

            def issue(j, buf, sem):
                pltpu.async_copy(
                    tbl.at[idxv.at[pl.ds(j * CH * L, CH * L)]], buf, sem)

            def wait(buf, sem):
                pltpu.make_async_copy(
                    tbl.at[idxv.at[pl.ds(0, CH * L)]], buf, sem).wait()

            def accum(j, buf):
                for c in range(CH):
                    row = j * CH + c
                    init = tuple(buf[c * L, pl.ds(VL * kk, VL)]
                                 for kk in range(DK))

                    def body(l, carry):
                        return tuple(
                            carry[kk] + buf[c * L + l, pl.ds(VL * kk, VL)]
                            for kk in range(DK))

                    sums = lax.fori_loop(1, L, body, init)
                    for kk in range(DK):
                        sl = (row, pl.ds(VL * kk, VL))
                        if t == 0:
                            acc[sl] = sums[kk]
                        else:
                            acc[sl] = acc[sl] + sums[kk]

            issue(0, rowsA, semA)

            @pl.loop(0, NCHUNK - 2, step=2)
            def _(jj):
                issue(jj + 1, rowsB, semB)
                wait(rowsA, semA)
                accum(jj, rowsA)
                issue(jj + 2, rowsA, semA)
                wait(rowsB, semB)
                accum(jj + 1, rowsB)

            issue(NCHUNK - 1, rowsB, semB)
            wait(rowsA, semA)
            accum(NCHUNK - 2, rowsA)
            wait(rowsB, semB)
            accum(NCHUNK - 1, rowsB)

        pltpu.sync_copy(acc, out_h.at[pl.ds(base, BPW), :])

    return k(x0, x1, x2, w0, w1, w2)


RELAYOUT_BLK = 2048


def _tc_relayout(w):
    V = w.shape[0]
    wt = w.T

    def body(wt_ref, o_ref):
        ident = (jax.lax.broadcasted_iota(jnp.int32, (D, D), 0) ==
                 jax.lax.broadcasted_iota(jnp.int32, (D, D), 1)
                 ).astype(jnp.float32)
        w = wt_ref[...]
        w_hi = w.astype(jnp.bfloat16).astype(jnp.float32)
        w_lo = w - w_hi
        dims = (((0,), (0,)), ((), ()))
        tt = (jax.lax.dot_general(w_hi, ident, dims,
                                  preferred_element_type=jnp.float32)
              + jax.lax.dot_general(w_lo, ident, dims,
                                    preferred_element_type=jnp.float32))
        t3 = tt.reshape(RELAYOUT_BLK // 2, 2, D)
        a = t3[:, 0, :]
        b = t3[:, 1, :]
        o_ref[...] = jnp.concatenate([a, b], axis=1)

    out = pl.pallas_call(
        body,
        grid=(pl.cdiv(V, RELAYOUT_BLK),),
        in_specs=[pl.BlockSpec((D, RELAYOUT_BLK), lambda j: (0, j))],
        out_specs=pl.BlockSpec((RELAYOUT_BLK // 2, 2 * D), lambda j: (j, 0)),
        out_shape=jax.ShapeDtypeStruct((V // 2, 2 * D), jnp.float32),
    )(wt)
    return out.reshape(V, D)


def _tc_mlp_softmax(e_sum, w_h, b_h, w_o, b_o):
    def body(e_ref, wh_ref, bh_ref, wo_ref, bo_ref, o_ref):
        wo = wo_ref[...]
        wc = jnp.dot(wh_ref[...], wo, preferred_element_type=jnp.float32)
        bc = jnp.dot(bh_ref[...], wo, preferred_element_type=jnp.float32) \
            + bo_ref[...]
        e_avg = e_ref[...] * (1.0 / (3 * L))
        logits = jnp.dot(e_avg, wc, preferred_element_type=jnp.float32) + bc
        m = jnp.max(logits, axis=1, keepdims=True)
        ex = jnp.exp(logits - m)
        o_ref[...] = ex / jnp.sum(ex, axis=1, keepdims=True)

    return pl.pallas_call(
        body,
        out_shape=jax.ShapeDtypeStruct((B, b_o.shape[-1]), jnp.float32),
    )(e_sum, w_h, b_h, w_o, b_o)


def kernel(x_0, x_1, x_2, W_word, W_2gram, W_3gram, W_h, b_h, W_o, b_o):
    x_0 = x_0.astype(jnp.int32).reshape(-1)
    x_1 = x_1.astype(jnp.int32).reshape(-1)
    x_2 = x_2.astype(jnp.int32).reshape(-1)
    W_word, W_2gram, W_3gram = [
        _tc_relayout(w) for w in (W_word, W_2gram, W_3gram)]
    e_sum = _sc_gather_sum(x_0, x_1, x_2, W_word, W_2gram, W_3gram)
    return _tc_mlp_softmax(e_sum, W_h.astype(jnp.float32),
                           b_h.reshape(1, -1).astype(jnp.float32),
                           W_o.astype(jnp.float32),
                           b_o.reshape(1, -1).astype(jnp.float32))


# speedup vs baseline: 1.4129x; 1.2446x over previous
import functools

import jax
import jax.numpy as jnp
from jax import lax
from jax.experimental import pallas as pl
from jax.experimental.pallas import tpu as pltpu
from jax.experimental.pallas import tpu_sc as plsc

B = 4096
L = 50
D = 64
NC = 2
NS = 16
NW = NC * NS
BPW = B // NW
CH = 8
NCHUNK = BPW // CH
VL = 16
DK = D // VL


def _sc_gather_sum(x0, x1, x2, w0, w1, w2):
    mesh = plsc.VectorSubcoreMesh(core_axis_name="c", subcore_axis_name="s")

    @functools.partial(
        pl.kernel,
        out_type=jax.ShapeDtypeStruct((B, D), jnp.float32),
        mesh=mesh,
        scratch_types=[
            pltpu.VMEM((BPW * L,), jnp.int32),
            pltpu.VMEM((BPW * L,), jnp.int32),
            pltpu.VMEM((BPW * L,), jnp.int32),
            pltpu.VMEM((CH * L, D), jnp.float32),
            pltpu.VMEM((CH * L, D), jnp.float32),
            pltpu.VMEM((BPW, D), jnp.float32),
            pltpu.SemaphoreType.DMA,
            pltpu.SemaphoreType.DMA,
        ],
        compiler_params=pltpu.CompilerParams(use_tc_tiling_on_sc=False),
    )
    def k(x0h, x1h, x2h, w0h, w1h, w2h, out_h,
          idx0, idx1, idx2, rowsA, rowsB, acc, semA, semB):
        wid = lax.axis_index("s") * NC + lax.axis_index("c")
        base = wid * BPW

        pltpu.sync_copy(x0h.at[pl.ds(base * L, BPW * L)], idx0)
        pltpu.sync_copy(x1h.at[pl.ds(base * L, BPW * L)], idx1)
        pltpu.sync_copy(x2h.at[pl.ds(base * L, BPW * L)], idx2)

        for t, (tbl, idxv) in enumerate(((w0h, idx0), (w1h, idx1), (w2h, idx2))):

            def issue(j, buf, sem):
                pltpu.async_copy(
                    tbl.at[idxv.at[pl.ds(j * CH * L, CH * L)]], buf, sem)

            def wait(buf, sem):
                pltpu.make_async_copy(
                    tbl.at[idxv.at[pl.ds(0, CH * L)]], buf, sem).wait()

            def accum(j, buf):
                for c in range(CH):
                    row = j * CH + c
                    init = tuple(buf[c * L, pl.ds(VL * kk, VL)]
                                 for kk in range(DK))

                    def body(l, carry):
                        return tuple(
                            carry[kk] + buf[c * L + l, pl.ds(VL * kk, VL)]
                            for kk in range(DK))

                    sums = lax.fori_loop(1, L, body, init)
                    for kk in range(DK):
                        sl = (row, pl.ds(VL * kk, VL))
                        if t == 0:
                            acc[sl] = sums[kk]
                        else:
                            acc[sl] = acc[sl] + sums[kk]

            issue(0, rowsA, semA)

            @pl.loop(0, NCHUNK - 2, step=2)
            def _(jj):
                issue(jj + 1, rowsB, semB)
                wait(rowsA, semA)
                accum(jj, rowsA)
                issue(jj + 2, rowsA, semA)
                wait(rowsB, semB)
                accum(jj + 1, rowsB)

            issue(NCHUNK - 1, rowsB, semB)
            wait(rowsA, semA)
            accum(NCHUNK - 2, rowsA)
            wait(rowsB, semB)
            accum(NCHUNK - 1, rowsB)

        pltpu.sync_copy(acc, out_h.at[pl.ds(base, BPW), :])

    return k(x0, x1, x2, w0, w1, w2)


RELAYOUT_BLK = 8192


def _tc_relayout(w):
    V = w.shape[0]
    wt = w.T

    def body(wt_ref, o_ref):
        tt = wt_ref[...].T
        t3 = tt.reshape(RELAYOUT_BLK // 2, 2, D)
        a = t3[:, 0, :]
        b = t3[:, 1, :]
        o_ref[...] = jnp.concatenate([a, b], axis=1)

    out = pl.pallas_call(
        body,
        grid=(pl.cdiv(V, RELAYOUT_BLK),),
        in_specs=[pl.BlockSpec((D, RELAYOUT_BLK), lambda j: (0, j))],
        out_specs=pl.BlockSpec((RELAYOUT_BLK // 2, 2 * D), lambda j: (j, 0)),
        out_shape=jax.ShapeDtypeStruct((V // 2, 2 * D), jnp.float32),
    )(wt)
    return out.reshape(V, D)


def _tc_mlp_softmax(e_sum, w_h, b_h, w_o, b_o):
    def body(e_ref, wh_ref, bh_ref, wo_ref, bo_ref, o_ref):
        wo = wo_ref[...]
        wc = jnp.dot(wh_ref[...], wo, preferred_element_type=jnp.float32)
        bc = jnp.dot(bh_ref[...], wo, preferred_element_type=jnp.float32) \
            + bo_ref[...]
        e_avg = e_ref[...] * (1.0 / (3 * L))
        logits = jnp.dot(e_avg, wc, preferred_element_type=jnp.float32) + bc
        m = jnp.max(logits, axis=1, keepdims=True)
        ex = jnp.exp(logits - m)
        o_ref[...] = ex / jnp.sum(ex, axis=1, keepdims=True)

    return pl.pallas_call(
        body,
        out_shape=jax.ShapeDtypeStruct((B, b_o.shape[-1]), jnp.float32),
    )(e_sum, w_h, b_h, w_o, b_o)


def kernel(x_0, x_1, x_2, W_word, W_2gram, W_3gram, W_h, b_h, W_o, b_o):
    x_0 = x_0.astype(jnp.int32).reshape(-1)
    x_1 = x_1.astype(jnp.int32).reshape(-1)
    x_2 = x_2.astype(jnp.int32).reshape(-1)
    W_word, W_2gram, W_3gram = [
        _tc_relayout(w) for w in (W_word, W_2gram, W_3gram)]
    e_sum = _sc_gather_sum(x_0, x_1, x_2, W_word, W_2gram, W_3gram)
    return _tc_mlp_softmax(e_sum, W_h.astype(jnp.float32),
                           b_h.reshape(1, -1).astype(jnp.float32),
                           W_o.astype(jnp.float32),
                           b_o.reshape(1, -1).astype(jnp.float32))


# speedup vs baseline: 1.7929x; 1.2689x over previous
import functools

import jax
import jax.numpy as jnp
from jax import lax
from jax.experimental import pallas as pl
from jax.experimental.pallas import tpu as pltpu
from jax.experimental.pallas import tpu_sc as plsc

B = 4096
L = 50
D = 64
NC = 2
NS = 16
NW = NC * NS
BPW = B // NW
CH = 8
NCHUNK = BPW // CH
VL = 16
DK = D // VL


def _sc_gather_sum_one(x, w):
    mesh = plsc.VectorSubcoreMesh(core_axis_name="c", subcore_axis_name="s")

    @functools.partial(
        pl.kernel,
        out_type=jax.ShapeDtypeStruct((B, D), jnp.float32),
        mesh=mesh,
        scratch_types=[
            pltpu.VMEM((BPW * L,), jnp.int32),
            pltpu.VMEM((CH * L, D), jnp.float32),
            pltpu.VMEM((CH * L, D), jnp.float32),
            pltpu.VMEM((BPW, D), jnp.float32),
            pltpu.SemaphoreType.DMA,
            pltpu.SemaphoreType.DMA,
        ],
        compiler_params=pltpu.CompilerParams(use_tc_tiling_on_sc=False),
    )
    def k(xh, wh, out_h, idx, rowsA, rowsB, acc, semA, semB):
        wid = lax.axis_index("s") * NC + lax.axis_index("c")
        base = wid * BPW

        pltpu.sync_copy(xh.at[pl.ds(base * L, BPW * L)], idx)

        def issue(j, buf, sem):
            pltpu.async_copy(
                wh.at[idx.at[pl.ds(j * CH * L, CH * L)]], buf, sem)

        def wait(buf, sem):
            pltpu.make_async_copy(
                wh.at[idx.at[pl.ds(0, CH * L)]], buf, sem).wait()

        def accum(j, buf):
            for c in range(CH):
                row = j * CH + c
                init = tuple(buf[c * L, pl.ds(VL * kk, VL)]
                             for kk in range(DK))

                def body(l, carry):
                    return tuple(
                        carry[kk] + buf[c * L + l, pl.ds(VL * kk, VL)]
                        for kk in range(DK))

                sums = lax.fori_loop(1, L, body, init)
                for kk in range(DK):
                    acc[row, pl.ds(VL * kk, VL)] = sums[kk]

        issue(0, rowsA, semA)

        @pl.loop(0, NCHUNK - 2, step=2)
        def _(jj):
            issue(jj + 1, rowsB, semB)
            wait(rowsA, semA)
            accum(jj, rowsA)
            issue(jj + 2, rowsA, semA)
            wait(rowsB, semB)
            accum(jj + 1, rowsB)

        issue(NCHUNK - 1, rowsB, semB)
        wait(rowsA, semA)
        accum(NCHUNK - 2, rowsA)
        wait(rowsB, semB)
        accum(NCHUNK - 1, rowsB)

        pltpu.sync_copy(acc, out_h.at[pl.ds(base, BPW), :])

    return k(x, w)


RELAYOUT_BLK = 8192


def _tc_relayout(w):
    V = w.shape[0]
    wt = w.T

    def body(wt_ref, o_ref):
        tt = wt_ref[...].T
        t3 = tt.reshape(RELAYOUT_BLK // 2, 2, D)
        a = t3[:, 0, :]
        b = t3[:, 1, :]
        o_ref[...] = jnp.concatenate([a, b], axis=1)

    out = pl.pallas_call(
        body,
        grid=(pl.cdiv(V, RELAYOUT_BLK),),
        in_specs=[pl.BlockSpec((D, RELAYOUT_BLK), lambda j: (0, j))],
        out_specs=pl.BlockSpec((RELAYOUT_BLK // 2, 2 * D), lambda j: (j, 0)),
        out_shape=jax.ShapeDtypeStruct((V // 2, 2 * D), jnp.float32),
    )(wt)
    return out.reshape(V, D)


def _tc_mlp_softmax(e0, e1, e2, w_h, b_h, w_o, b_o):
    def body(e0_ref, e1_ref, e2_ref, wh_ref, bh_ref, wo_ref, bo_ref, o_ref):
        wo = wo_ref[...]
        wc = jnp.dot(wh_ref[...], wo, preferred_element_type=jnp.float32)
        bc = jnp.dot(bh_ref[...], wo, preferred_element_type=jnp.float32) \
            + bo_ref[...]
        e_avg = (e0_ref[...] + e1_ref[...] + e2_ref[...]) * (1.0 / (3 * L))
        logits = jnp.dot(e_avg, wc, preferred_element_type=jnp.float32) + bc
        m = jnp.max(logits, axis=1, keepdims=True)
        ex = jnp.exp(logits - m)
        o_ref[...] = ex / jnp.sum(ex, axis=1, keepdims=True)

    return pl.pallas_call(
        body,
        out_shape=jax.ShapeDtypeStruct((B, b_o.shape[-1]), jnp.float32),
    )(e0, e1, e2, w_h, b_h, w_o, b_o)


def kernel(x_0, x_1, x_2, W_word, W_2gram, W_3gram, W_h, b_h, W_o, b_o):
    xs = [x.astype(jnp.int32).reshape(-1) for x in (x_0, x_1, x_2)]
    ws = [_tc_relayout(w) for w in (W_word, W_2gram, W_3gram)]
    es = [_sc_gather_sum_one(x, w) for x, w in zip(xs, ws)]
    return _tc_mlp_softmax(*es, W_h.astype(jnp.float32),
                           b_h.reshape(1, -1).astype(jnp.float32),
                           W_o.astype(jnp.float32),
                           b_o.reshape(1, -1).astype(jnp.float32))
